# baseline (speedup 1.0000x reference)
import functools

import jax
import jax.numpy as jnp
from jax import lax
from jax.experimental import pallas as pl
from jax.experimental.pallas import tpu as pltpu
from jax.experimental.pallas import tpu_sc as plsc

EMB = 128
SC_CORES = 2
SC_SUBCORES = 16
SC_WORKERS = SC_CORES * SC_SUBCORES
GATHER_CHUNK = 128

MLP_TILE = 4096


def _sc_gather_pair(users, items, user_table, item_table):
    batch = users.shape[0]
    per_worker = batch // SC_WORKERS
    n_chunks = per_worker // GATHER_CHUNK
    mesh = plsc.VectorSubcoreMesh(core_axis_name="c", subcore_axis_name="s")

    @functools.partial(
        pl.kernel,
        mesh=mesh,
        out_type=(
            jax.ShapeDtypeStruct((batch, EMB), user_table.dtype),
            jax.ShapeDtypeStruct((batch, EMB), item_table.dtype),
        ),
        scratch_types=[
            pltpu.VMEM((GATHER_CHUNK,), jnp.int32),
            pltpu.VMEM((GATHER_CHUNK,), jnp.int32),
            pltpu.VMEM((GATHER_CHUNK,), jnp.int32),
            pltpu.VMEM((GATHER_CHUNK,), jnp.int32),
            pltpu.VMEM((GATHER_CHUNK, EMB), jnp.float32),
            pltpu.VMEM((GATHER_CHUNK, EMB), jnp.float32),
            pltpu.VMEM((GATHER_CHUNK, EMB), jnp.float32),
            pltpu.VMEM((GATHER_CHUNK, EMB), jnp.float32),
            pltpu.SemaphoreType.DMA,
            pltpu.SemaphoreType.DMA,
            pltpu.SemaphoreType.DMA,
            pltpu.SemaphoreType.DMA,
            pltpu.SemaphoreType.DMA,
            pltpu.SemaphoreType.DMA,
            pltpu.SemaphoreType.DMA,
            pltpu.SemaphoreType.DMA,
        ],
    )
    def gather_kernel(ut_hbm, it_hbm, u_idx_hbm, i_idx_hbm, ou_hbm, oi_hbm,
                      iu0, iu1, ii0, ii1,
                      ru0, ru1, ri0, ri1,
                      gu0, gu1, gi0, gi1, wu0, wu1, wi0, wi1):
        wid = lax.axis_index("s") * SC_CORES + lax.axis_index("c")
        base = wid * per_worker
        idx_u, idx_i = (iu0, iu1), (ii0, ii1)
        rows_u, rows_i = (ru0, ru1), (ri0, ri1)
        gsem_u, gsem_i = (gu0, gu1), (gi0, gi1)
        wsem_u, wsem_i = (wu0, wu1), (wi0, wi1)
        writes = [None] * n_chunks
        for p in range(n_chunks):
            b = p & 1
            off = base + p * GATHER_CHUNK
            if p >= 2:
                writes[p - 2][0].wait()
                writes[p - 2][1].wait()
            pltpu.sync_copy(u_idx_hbm.at[pl.ds(off, GATHER_CHUNK)], idx_u[b])
            pltpu.sync_copy(i_idx_hbm.at[pl.ds(off, GATHER_CHUNK)], idx_i[b])
            cu = pltpu.async_copy(ut_hbm.at[idx_u[b]], rows_u[b], gsem_u[b])
            ci = pltpu.async_copy(it_hbm.at[idx_i[b]], rows_i[b], gsem_i[b])
            cu.wait()
            wu = pltpu.async_copy(rows_u[b], ou_hbm.at[pl.ds(off, GATHER_CHUNK)],
                                  wsem_u[b])
            ci.wait()
            wi = pltpu.async_copy(rows_i[b], oi_hbm.at[pl.ds(off, GATHER_CHUNK)],
                                  wsem_i[b])
            writes[p] = (wu, wi)
        for p in range(max(n_chunks - 2, 0), n_chunks):
            writes[p][0].wait()
            writes[p][1].wait()

    return gather_kernel(user_table, item_table, users, items)


_CONTRACT_LAST = (((1,), (1,)), ((), ()))


def _mlp_body(u_ref, i_ref, w1u_ref, w1i_ref, b1_ref, w2_ref, b2_ref,
              wf_ref, bf_ref, o_ref):
    u = u_ref[...].astype(jnp.bfloat16)
    i = i_ref[...].astype(jnp.bfloat16)
    x = lax.dot_general(u, w1u_ref[...], _CONTRACT_LAST,
                        preferred_element_type=jnp.float32)
    x = x + lax.dot_general(i, w1i_ref[...], _CONTRACT_LAST,
                            preferred_element_type=jnp.float32)
    h1 = jnp.maximum(x + b1_ref[...], 0.0).astype(jnp.bfloat16)
    h2 = jnp.maximum(
        lax.dot_general(h1, w2_ref[...], _CONTRACT_LAST,
                        preferred_element_type=jnp.float32)
        + b2_ref[...], 0.0)
    z = lax.dot_general(wf_ref[...], h2, _CONTRACT_LAST,
                        preferred_element_type=jnp.float32) + bf_ref[...]
    o_ref[...] = jax.nn.sigmoid(z)


def _tc_mlp(u_emb, i_emb, w1u_t, w1i_t, b1, w2_t, b2, wf, bf):
    batch = u_emb.shape[0]
    grid = (batch // MLP_TILE,)
    emb_spec = pl.BlockSpec((MLP_TILE, EMB), lambda i: (i, 0))
    full = lambda shape: pl.BlockSpec(shape, lambda i: (0, 0))
    return pl.pallas_call(
        _mlp_body,
        grid=grid,
        in_specs=[
            emb_spec,
            emb_spec,
            full((256, EMB)),
            full((256, EMB)),
            full((1, 256)),
            full((EMB, 256)),
            full((1, EMB)),
            full((1, EMB)),
            full((1, 1)),
        ],
        out_specs=pl.BlockSpec((1, MLP_TILE), lambda i: (0, i)),
        out_shape=jax.ShapeDtypeStruct((1, batch), jnp.float32),
        compiler_params=pltpu.CompilerParams(
            dimension_semantics=("parallel",)),
    )(u_emb, i_emb, w1u_t, w1i_t, b1, w2_t, b2, wf, bf)


def kernel(users, items, user_table, item_table, W1, b1, W2, b2, Wf, bf):
    u_emb, i_emb = _sc_gather_pair(users, items, user_table, item_table)
    w1u = W1[:, :EMB].astype(jnp.bfloat16)
    w1i = W1[:, EMB:].astype(jnp.bfloat16)
    out_row = _tc_mlp(u_emb, i_emb, w1u, w1i,
                      b1.reshape(1, 256), W2.astype(jnp.bfloat16),
                      b2.reshape(1, EMB), Wf, bf.reshape(1, 1))
    return out_row.reshape(users.shape[0], 1)


# speedup vs baseline: 1.0521x; 1.0521x over previous
import functools

import jax
import jax.numpy as jnp
from jax import lax
from jax.experimental import pallas as pl
from jax.experimental.pallas import tpu as pltpu
from jax.experimental.pallas import tpu_sc as plsc

EMB = 128
SC_CORES = 2
SC_SUBCORES = 16
SC_WORKERS = SC_CORES * SC_SUBCORES
GATHER_CHUNK = 128

MLP_TILE = 4096


def _sc_gather_pair(users, items, user_table, item_table):
    batch = users.shape[0]
    per_worker = batch // SC_WORKERS
    n_chunks = per_worker // GATHER_CHUNK
    mesh = plsc.VectorSubcoreMesh(core_axis_name="c", subcore_axis_name="s")

    @functools.partial(
        pl.kernel,
        mesh=mesh,
        out_type=(
            jax.ShapeDtypeStruct((batch, EMB), user_table.dtype),
            jax.ShapeDtypeStruct((batch, EMB), item_table.dtype),
        ),
        scratch_types=[
            pltpu.VMEM((GATHER_CHUNK,), jnp.int32),
            pltpu.VMEM((GATHER_CHUNK,), jnp.int32),
            pltpu.VMEM((GATHER_CHUNK,), jnp.int32),
            pltpu.VMEM((GATHER_CHUNK,), jnp.int32),
            pltpu.VMEM((GATHER_CHUNK, EMB), jnp.float32),
            pltpu.VMEM((GATHER_CHUNK, EMB), jnp.float32),
            pltpu.VMEM((GATHER_CHUNK, EMB), jnp.float32),
            pltpu.VMEM((GATHER_CHUNK, EMB), jnp.float32),
            pltpu.SemaphoreType.DMA,
            pltpu.SemaphoreType.DMA,
            pltpu.SemaphoreType.DMA,
            pltpu.SemaphoreType.DMA,
            pltpu.SemaphoreType.DMA,
            pltpu.SemaphoreType.DMA,
            pltpu.SemaphoreType.DMA,
            pltpu.SemaphoreType.DMA,
        ],
    )
    def gather_kernel(ut_hbm, it_hbm, u_idx_hbm, i_idx_hbm, ou_hbm, oi_hbm,
                      iu0, iu1, ii0, ii1,
                      ru0, ru1, ri0, ri1,
                      gu0, gu1, gi0, gi1, wu0, wu1, wi0, wi1):
        wid = lax.axis_index("s") * SC_CORES + lax.axis_index("c")
        base = wid * per_worker
        idx = (iu0, iu1, ii0, ii1)
        rows = (ru0, ru1, ri0, ri1)
        gsem = (gu0, gu1, gi0, gi1)
        wsem = (wu0, wu1, wi0, wi1)
        n_steps = per_worker // (2 * GATHER_CHUNK)
        writes = [None] * 4
        for s in range(n_steps):
            off = base + s * 2 * GATHER_CHUNK
            gathers = []
            for k in range(4):
                src = u_idx_hbm if k < 2 else i_idx_hbm
                koff = off + (k & 1) * GATHER_CHUNK
                if s > 0:
                    writes[k].wait()
                pltpu.sync_copy(src.at[pl.ds(koff, GATHER_CHUNK)], idx[k])
                table = ut_hbm if k < 2 else it_hbm
                gathers.append(
                    pltpu.async_copy(table.at[idx[k]], rows[k], gsem[k]))
            for k in range(4):
                dst = ou_hbm if k < 2 else oi_hbm
                koff = off + (k & 1) * GATHER_CHUNK
                gathers[k].wait()
                writes[k] = pltpu.async_copy(
                    rows[k], dst.at[pl.ds(koff, GATHER_CHUNK)], wsem[k])
        for k in range(4):
            writes[k].wait()

    return gather_kernel(user_table, item_table, users, items)


_CONTRACT_LAST = (((1,), (1,)), ((), ()))


def _mlp_body(u_ref, i_ref, w1u_ref, w1i_ref, b1_ref, w2_ref, b2_ref,
              wf_ref, bf_ref, o_ref):
    u = u_ref[...].astype(jnp.bfloat16)
    i = i_ref[...].astype(jnp.bfloat16)
    x = lax.dot_general(u, w1u_ref[...], _CONTRACT_LAST,
                        preferred_element_type=jnp.float32)
    x = x + lax.dot_general(i, w1i_ref[...], _CONTRACT_LAST,
                            preferred_element_type=jnp.float32)
    h1 = jnp.maximum(x + b1_ref[...], 0.0).astype(jnp.bfloat16)
    h2 = jnp.maximum(
        lax.dot_general(h1, w2_ref[...], _CONTRACT_LAST,
                        preferred_element_type=jnp.float32)
        + b2_ref[...], 0.0)
    z = lax.dot_general(wf_ref[...], h2, _CONTRACT_LAST,
                        preferred_element_type=jnp.float32) + bf_ref[...]
    o_ref[...] = jax.nn.sigmoid(z)


def _tc_mlp(u_emb, i_emb, w1u_t, w1i_t, b1, w2_t, b2, wf, bf):
    batch = u_emb.shape[0]
    grid = (batch // MLP_TILE,)
    emb_spec = pl.BlockSpec((MLP_TILE, EMB), lambda i: (i, 0))
    full = lambda shape: pl.BlockSpec(shape, lambda i: (0, 0))
    return pl.pallas_call(
        _mlp_body,
        grid=grid,
        in_specs=[
            emb_spec,
            emb_spec,
            full((256, EMB)),
            full((256, EMB)),
            full((1, 256)),
            full((EMB, 256)),
            full((1, EMB)),
            full((1, EMB)),
            full((1, 1)),
        ],
        out_specs=pl.BlockSpec((1, MLP_TILE), lambda i: (0, i)),
        out_shape=jax.ShapeDtypeStruct((1, batch), jnp.float32),
        compiler_params=pltpu.CompilerParams(
            dimension_semantics=("arbitrary",)),
    )(u_emb, i_emb, w1u_t, w1i_t, b1, w2_t, b2, wf, bf)


def kernel(users, items, user_table, item_table, W1, b1, W2, b2, Wf, bf):
    u_emb, i_emb = _sc_gather_pair(users, items, user_table, item_table)
    w1u = W1[:, :EMB].astype(jnp.bfloat16)
    w1i = W1[:, EMB:].astype(jnp.bfloat16)
    out_row = _tc_mlp(u_emb, i_emb, w1u, w1i,
                      b1.reshape(1, 256), W2.astype(jnp.bfloat16),
                      b2.reshape(1, EMB), Wf, bf.reshape(1, 1))
    return out_row.reshape(users.shape[0], 1)
